# Initial kernel scaffold; baseline (speedup 1.0000x reference)
import jax
import jax.numpy as jnp
from jax.experimental import pallas as pl


def kernel(x, attention_mask, emb_table, W1, b1, W2, b2):
    raise NotImplementedError("write your pallas kernel here")


# speedup vs baseline: 20.0879x; 20.0879x over previous
import functools

import jax
import jax.numpy as jnp
from jax import lax
from jax.experimental import pallas as pl
from jax.experimental.pallas import tpu as pltpu
from jax.experimental.pallas import tpu_sc as plsc

VOCAB = 100000
EMB_DIM = 128
B, L = 4096, 200

GRID_T = 25
ROWS_T = VOCAB // GRID_T

NC, NS = 2, 16
NW = NC * NS
TOK_W = (B * L) // NW
ROWS_W = B // NW
GROUPS_W = ROWS_W // 16


def _tdot_body(w1_ref, w2_ref, tbl_ref, t_ref):
    v = jnp.dot(w2_ref[...], w1_ref[...], preferred_element_type=jnp.float32)
    blk = tbl_ref[0]
    t_ref[0, 0, :] = jnp.sum(blk * v, axis=-1)


def _table_dot(emb_table, W1, W2):
    t3 = pl.pallas_call(
        _tdot_body,
        grid=(GRID_T,),
        in_specs=[
            pl.BlockSpec((64, EMB_DIM), lambda i: (0, 0)),
            pl.BlockSpec((1, 64), lambda i: (0, 0)),
            pl.BlockSpec((1, ROWS_T, EMB_DIM), lambda i: (i, 0, 0)),
        ],
        out_specs=pl.BlockSpec((1, 1, ROWS_T), lambda i: (i, 0, 0)),
        out_shape=jax.ShapeDtypeStruct((GRID_T, 1, ROWS_T), jnp.float32),
    )(W1, W2, emb_table.reshape(GRID_T, ROWS_T, EMB_DIM))
    return t3.reshape(VOCAB)


_SC_MESH = plsc.VectorSubcoreMesh(
    core_axis_name="c", subcore_axis_name="s", num_cores=NC, num_subcores=NS
)


@functools.partial(
    pl.kernel,
    out_type=jax.ShapeDtypeStruct((B,), jnp.float32),
    mesh=_SC_MESH,
    compiler_params=pltpu.CompilerParams(needs_layout_passes=False),
    scratch_types=[
        pltpu.VMEM((TOK_W,), jnp.int32),
        pltpu.VMEM((TOK_W,), jnp.float32),
        pltpu.VMEM((TOK_W,), jnp.float32),
        pltpu.VMEM((ROWS_W,), jnp.float32),
        pltpu.VMEM((16,), jnp.float32),
        pltpu.SemaphoreType.DMA,
    ],
)
def _sc_pool(t_hbm, x_hbm, m_hbm, c_hbm, out_hbm, idx_v, val_v, msk_v, o_v, c_v, sem):
    wid = lax.axis_index("s") * NC + lax.axis_index("c")
    tbase = wid * TOK_W
    pltpu.sync_copy(x_hbm.at[pl.ds(tbase, TOK_W)], idx_v)
    pltpu.sync_copy(m_hbm.at[pl.ds(tbase, TOK_W)], msk_v)
    pltpu.sync_copy(c_hbm, c_v)
    pltpu.async_copy(t_hbm.at[idx_v], val_v, sem).wait()

    lanes = lax.broadcasted_iota(jnp.int32, (16,), 0)
    c16 = c_v[...]
    one = jnp.ones((16,), jnp.float32)

    for g in range(GROUPS_W):
        base16 = (g * 16 + lanes) * L

        def body(k, carry):
            acc, mac = carry
            i16 = base16 + k
            v16 = plsc.load_gather(val_v, [i16])
            m16 = plsc.load_gather(msk_v, [i16])
            return acc + v16 * m16, mac + m16

        acc, mac = lax.fori_loop(
            0, L, body,
            (jnp.zeros((16,), jnp.float32), jnp.zeros((16,), jnp.float32)),
        )
        z = acc / jnp.maximum(mac, one) + c16
        o_v[pl.ds(g * 16, 16)] = one / (one + jnp.exp(-z))

    pltpu.sync_copy(o_v, out_hbm.at[pl.ds(wid * ROWS_W, ROWS_W)])


def kernel(x, attention_mask, emb_table, W1, b1, W2, b2):
    t = _table_dot(emb_table, W1, W2)
    c = jnp.dot(W2[0], b1) + b2[0]
    c_vec = jnp.full((16,), c, jnp.float32)
    x_flat = x.reshape(-1).astype(jnp.int32)
    m_flat = attention_mask.reshape(-1).astype(jnp.float32)
    return _sc_pool(t, x_flat, m_flat, c_vec)


# speedup vs baseline: 27.4582x; 1.3669x over previous
import functools

import jax
import jax.numpy as jnp
from jax import lax
from jax.experimental import pallas as pl
from jax.experimental.pallas import tpu as pltpu
from jax.experimental.pallas import tpu_sc as plsc

VOCAB = 100000
EMB_DIM = 128
B, L = 4096, 200

GRID_T = 25
ROWS_T = VOCAB // GRID_T

NC, NS = 2, 16
NW = NC * NS
TOK_W = (B * L) // NW
ROWS_W = B // NW
GROUPS_W = ROWS_W // 16
UNROLL = 8
T_PAD = VOCAB + 16


def _tdot_body(w1_ref, w2_ref, tbl_ref, t_ref):
    v = jnp.dot(w2_ref[...], w1_ref[...], preferred_element_type=jnp.float32)
    t_ref[0, 0, :] = jnp.dot(tbl_ref[...], v[0], preferred_element_type=jnp.float32)


def _table_dot(emb_table, W1, W2):
    t3 = pl.pallas_call(
        _tdot_body,
        grid=(GRID_T,),
        in_specs=[
            pl.BlockSpec((64, EMB_DIM), lambda i: (0, 0)),
            pl.BlockSpec((1, 64), lambda i: (0, 0)),
            pl.BlockSpec((ROWS_T, EMB_DIM), lambda i: (i, 0)),
        ],
        out_specs=pl.BlockSpec((1, 1, ROWS_T), lambda i: (i, 0, 0)),
        out_shape=jax.ShapeDtypeStruct((GRID_T, 1, ROWS_T), jnp.float32),
    )(W1, W2, emb_table)
    return t3.reshape(VOCAB)


_SC_MESH = plsc.VectorSubcoreMesh(
    core_axis_name="c", subcore_axis_name="s", num_cores=NC, num_subcores=NS
)


@functools.partial(
    pl.kernel,
    out_type=jax.ShapeDtypeStruct((B,), jnp.float32),
    mesh=_SC_MESH,
    compiler_params=pltpu.CompilerParams(needs_layout_passes=False),
    scratch_types=[
        pltpu.VMEM((T_PAD,), jnp.float32),
        pltpu.VMEM((TOK_W,), jnp.int32),
        pltpu.VMEM((ROWS_W,), jnp.float32),
        pltpu.VMEM((16,), jnp.float32),
    ],
)
def _sc_pool(t_hbm, xm_hbm, c_hbm, out_hbm, t_v, idx_v, o_v, c_v):
    wid = lax.axis_index("s") * NC + lax.axis_index("c")
    zero16 = jnp.zeros((16,), jnp.float32)
    one16 = jnp.ones((16,), jnp.float32)
    pltpu.sync_copy(t_hbm, t_v.at[pl.ds(0, VOCAB)])
    t_v[pl.ds(VOCAB, 16)] = zero16
    pltpu.sync_copy(xm_hbm.at[pl.ds(wid * TOK_W, TOK_W)], idx_v)
    pltpu.sync_copy(c_hbm, c_v)

    lanes = lax.broadcasted_iota(jnp.int32, (16,), 0)
    c16 = c_v[...]
    sink16 = jnp.full((16,), VOCAB, jnp.int32)

    for g in range(GROUPS_W):
        goff16 = (g * 16 + lanes) * L

        def body(kk, carry, goff16=goff16):
            acc, mac = carry
            for u in range(UNROLL):
                xi = plsc.load_gather(idx_v, [goff16 + (kk * UNROLL + u)])
                acc = acc + plsc.load_gather(t_v, [xi])
                mac = mac + jnp.where(xi < sink16, one16, zero16)
            return acc, mac

        acc, mac = lax.fori_loop(0, L // UNROLL, body, (zero16, zero16))
        z = acc / jnp.maximum(mac, one16) + c16
        o_v[pl.ds(g * 16, 16)] = one16 / (one16 + jnp.exp(-z))

    pltpu.sync_copy(o_v, out_hbm.at[pl.ds(wid * ROWS_W, ROWS_W)])


def kernel(x, attention_mask, emb_table, W1, b1, W2, b2):
    t = _table_dot(emb_table, W1, W2)
    c = jnp.dot(W2[0], b1) + b2[0]
    c_vec = jnp.full((16,), c, jnp.float32)
    xm = jnp.where(attention_mask != 0, x.astype(jnp.int32), VOCAB).reshape(-1)
    return _sc_pool(t, xm, c_vec)


# speedup vs baseline: 28.0657x; 1.0221x over previous
import functools

import jax
import jax.numpy as jnp
from jax import lax
from jax.experimental import pallas as pl
from jax.experimental.pallas import tpu as pltpu
from jax.experimental.pallas import tpu_sc as plsc

VOCAB = 100000
EMB_DIM = 128
B, L = 4096, 200

GRID_T = 10
ROWS_T = VOCAB // GRID_T

NC, NS = 2, 16
NW = NC * NS
TOK_W = (B * L) // NW
ROWS_W = B // NW
GROUPS_W = ROWS_W // 16
UNROLL = 8
T_PAD = VOCAB + 16


def _tdot_body(w1_ref, w2_ref, tbl_ref, t_ref):
    v = jnp.dot(w2_ref[...], w1_ref[...], preferred_element_type=jnp.float32)
    t_ref[0, 0, :] = jnp.dot(tbl_ref[...], v[0], preferred_element_type=jnp.float32)


def _table_dot(emb_table, W1, W2):
    t3 = pl.pallas_call(
        _tdot_body,
        grid=(GRID_T,),
        in_specs=[
            pl.BlockSpec((64, EMB_DIM), lambda i: (0, 0)),
            pl.BlockSpec((1, 64), lambda i: (0, 0)),
            pl.BlockSpec((ROWS_T, EMB_DIM), lambda i: (i, 0)),
        ],
        out_specs=pl.BlockSpec((1, 1, ROWS_T), lambda i: (i, 0, 0)),
        out_shape=jax.ShapeDtypeStruct((GRID_T, 1, ROWS_T), jnp.float32),
    )(W1, W2, emb_table)
    return t3.reshape(VOCAB)


_SC_MESH = plsc.VectorSubcoreMesh(
    core_axis_name="c", subcore_axis_name="s", num_cores=NC, num_subcores=NS
)


@functools.partial(
    pl.kernel,
    out_type=jax.ShapeDtypeStruct((B,), jnp.float32),
    mesh=_SC_MESH,
    compiler_params=pltpu.CompilerParams(needs_layout_passes=False),
    scratch_types=[
        pltpu.VMEM((T_PAD,), jnp.float32),
        pltpu.VMEM((TOK_W,), jnp.int32),
        pltpu.VMEM((ROWS_W,), jnp.float32),
        pltpu.VMEM((16,), jnp.float32),
    ],
)
def _sc_pool(t_hbm, xm_hbm, c_hbm, out_hbm, t_v, idx_v, o_v, c_v):
    wid = lax.axis_index("s") * NC + lax.axis_index("c")
    zero16 = jnp.zeros((16,), jnp.float32)
    one16 = jnp.ones((16,), jnp.float32)
    pltpu.sync_copy(t_hbm, t_v.at[pl.ds(0, VOCAB)])
    t_v[pl.ds(VOCAB, 16)] = zero16
    pltpu.sync_copy(xm_hbm.at[pl.ds(wid * TOK_W, TOK_W)], idx_v)
    pltpu.sync_copy(c_hbm, c_v)

    lanes = lax.broadcasted_iota(jnp.int32, (16,), 0)
    c16 = c_v[...]
    sink16 = jnp.full((16,), VOCAB, jnp.int32)

    for g in range(GROUPS_W):
        goff16 = (g * 16 + lanes) * L

        def body(kk, carry, goff16=goff16):
            acc, mac = carry
            for u in range(UNROLL):
                xi = plsc.load_gather(idx_v, [goff16 + (kk * UNROLL + u)])
                acc = acc + plsc.load_gather(t_v, [xi])
                mac = mac + jnp.where(xi < sink16, one16, zero16)
            return acc, mac

        acc, mac = lax.fori_loop(0, L // UNROLL, body, (zero16, zero16))
        z = acc / jnp.maximum(mac, one16) + c16
        o_v[pl.ds(g * 16, 16)] = one16 / (one16 + jnp.exp(-z))

    pltpu.sync_copy(o_v, out_hbm.at[pl.ds(wid * ROWS_W, ROWS_W)])


def kernel(x, attention_mask, emb_table, W1, b1, W2, b2):
    t = _table_dot(emb_table, W1, W2)
    c = jnp.dot(W2[0], b1) + b2[0]
    c_vec = jnp.full((16,), c, jnp.float32)
    xm = jnp.where(attention_mask != 0, x.astype(jnp.int32), VOCAB).reshape(-1)
    return _sc_pool(t, xm, c_vec)


# speedup vs baseline: 38.1096x; 1.3579x over previous
import functools

import jax
import jax.numpy as jnp
from jax import lax
from jax.experimental import pallas as pl
from jax.experimental.pallas import tpu as pltpu
from jax.experimental.pallas import tpu_sc as plsc

VOCAB = 100000
EMB_DIM = 128
B, L = 4096, 200

GRID_T = 10
ROWS_T = VOCAB // GRID_T

NC, NS = 2, 16
NW = NC * NS
TOK_W = (B * L) // NW
ROWS_W = B // NW
GROUPS_W = ROWS_W // 16
UNROLL = 8
T_PAD = VOCAB + 16


def _tdot_body(w1_ref, w2_ref, tbl_ref, t_ref):
    v = jnp.dot(w2_ref[...], w1_ref[...], preferred_element_type=jnp.float32)
    t_ref[0, :, :] = lax.dot_general(
        v, tbl_ref[...], (((1,), (1,)), ((), ())),
        preferred_element_type=jnp.float32,
    )


def _table_dot(emb_table, W1, W2):
    t3 = pl.pallas_call(
        _tdot_body,
        grid=(GRID_T,),
        in_specs=[
            pl.BlockSpec((64, EMB_DIM), lambda i: (0, 0)),
            pl.BlockSpec((1, 64), lambda i: (0, 0)),
            pl.BlockSpec((ROWS_T, EMB_DIM), lambda i: (i, 0)),
        ],
        out_specs=pl.BlockSpec((1, 1, ROWS_T), lambda i: (i, 0, 0)),
        out_shape=jax.ShapeDtypeStruct((GRID_T, 1, ROWS_T), jnp.float32),
    )(W1, W2, emb_table)
    return t3.reshape(VOCAB)


_SC_MESH = plsc.VectorSubcoreMesh(
    core_axis_name="c", subcore_axis_name="s", num_cores=NC, num_subcores=NS
)


@functools.partial(
    pl.kernel,
    out_type=jax.ShapeDtypeStruct((B,), jnp.float32),
    mesh=_SC_MESH,
    compiler_params=pltpu.CompilerParams(needs_layout_passes=False),
    scratch_types=[
        pltpu.VMEM((T_PAD,), jnp.float32),
        pltpu.VMEM((TOK_W,), jnp.int32),
        pltpu.VMEM((ROWS_W,), jnp.float32),
        pltpu.VMEM((16,), jnp.float32),
    ],
)
def _sc_pool(t_hbm, xm_hbm, c_hbm, out_hbm, t_v, idx_v, o_v, c_v):
    wid = lax.axis_index("s") * NC + lax.axis_index("c")
    zero16 = jnp.zeros((16,), jnp.float32)
    one16 = jnp.ones((16,), jnp.float32)
    pltpu.sync_copy(t_hbm, t_v.at[pl.ds(0, VOCAB)])
    t_v[pl.ds(VOCAB, 16)] = zero16
    pltpu.sync_copy(xm_hbm.at[pl.ds(wid * TOK_W, TOK_W)], idx_v)
    pltpu.sync_copy(c_hbm, c_v)

    lanes = lax.broadcasted_iota(jnp.int32, (16,), 0)
    c16 = c_v[...]
    sink16 = jnp.full((16,), VOCAB, jnp.int32)

    for g in range(GROUPS_W):
        goff16 = (g * 16 + lanes) * L

        def body(kk, carry, goff16=goff16):
            acc, mac = carry
            for u in range(UNROLL):
                xi = plsc.load_gather(idx_v, [goff16 + (kk * UNROLL + u)])
                acc = acc + plsc.load_gather(t_v, [xi])
                mac = mac + jnp.where(xi < sink16, one16, zero16)
            return acc, mac

        acc, mac = lax.fori_loop(0, L // UNROLL, body, (zero16, zero16))
        z = acc / jnp.maximum(mac, one16) + c16
        o_v[pl.ds(g * 16, 16)] = one16 / (one16 + jnp.exp(-z))

    pltpu.sync_copy(o_v, out_hbm.at[pl.ds(wid * ROWS_W, ROWS_W)])


def kernel(x, attention_mask, emb_table, W1, b1, W2, b2):
    t = _table_dot(emb_table, W1, W2)
    c = jnp.dot(W2[0], b1) + b2[0]
    c_vec = jnp.full((16,), c, jnp.float32)
    xm = jnp.where(attention_mask != 0, x.astype(jnp.int32), VOCAB).reshape(-1)
    return _sc_pool(t, xm, c_vec)
